# baseline (speedup 1.0000x reference)
import functools

import jax
import jax.numpy as jnp
from jax import lax
from jax.experimental import pallas as pl
from jax.experimental.pallas import tpu as pltpu
from jax.experimental.pallas import tpu_sc as plsc

N_TOK = 8192
D = 1024
H = 2048
E = 8
BLK = 512
NB_MAX = N_TOK // BLK + E
NP_PAD = NB_MAX * BLK
GATE_CHUNK = 2048
NCHUNK = N_TOK // GATE_CHUNK
NEG = -1e30


def _route_body(x_ref, wg_ref, sel_ref, rank_ref, meta_ref, cnt, carry):
    c = pl.program_id(0)

    @pl.when(c == 0)
    def _():
        cnt[...] = jnp.zeros_like(cnt)
        carry[...] = jnp.zeros_like(carry)

    @pl.when(c < NCHUNK)
    def _():
        logits = lax.dot_general(x_ref[...], wg_ref[...],
                                 (((1,), (1,)), ((), ())),
                                 preferred_element_type=jnp.float32)
        mx = jnp.max(logits, axis=1, keepdims=True)
        lane = lax.broadcasted_iota(jnp.int32, (GATE_CHUNK, E), 1)
        bidx = jnp.min(jnp.where(logits == mx, lane, E),
                       axis=1, keepdims=True)
        sel_ref[...] = bidx.reshape(1, 1, GATE_CHUNK)
        sel2 = bidx.reshape(1, GATE_CHUNK)
        iota_e = lax.broadcasted_iota(jnp.int32, (E, GATE_CHUNK), 0)
        oht = (iota_e == sel2).astype(jnp.float32)
        scan = jnp.pad(oht[:, :-1], ((0, 0), (1, 0)))
        sh = 1
        while sh < GATE_CHUNK:
            scan = scan + jnp.pad(scan[:, :-sh], ((0, 0), (sh, 0)))
            sh *= 2
        rank = carry[:, 0:1] + scan
        rankf = jnp.sum(oht * rank, axis=0, keepdims=True)
        rank_ref[...] = rankf.astype(jnp.int32).reshape(1, 1, GATE_CHUNK)
        carry[:, 0:1] += jnp.sum(oht, axis=1, keepdims=True)
        cnt[:, 0:1] += jnp.sum(oht, axis=1, keepdims=True)

    @pl.when(c == NCHUNK)
    def _():
        counts = cnt[:, 0:1]
        nb = jnp.ceil(counts / BLK)
        tri = (lax.broadcasted_iota(jnp.int32, (E, E), 1)
               < lax.broadcasted_iota(jnp.int32, (E, E), 0)).astype(jnp.float32)
        ao = jnp.sum(tri * jnp.transpose(nb), axis=1, keepdims=True) * BLK
        blk_start = ao / BLK
        bidx = lax.broadcasted_iota(jnp.int32, (1, 128), 1).astype(jnp.float32)
        inb = bidx >= blk_start
        own = inb & (bidx < blk_start + nb)
        ownf = own.astype(jnp.float32)
        ve = jnp.clip(counts - (bidx - blk_start) * BLK, 0.0, float(BLK))
        bv = jnp.sum(ownf * ve, axis=0, keepdims=True)
        iota_col = lax.broadcasted_iota(jnp.int32, (E, 1), 0).astype(jnp.float32)
        be_own = jnp.sum(ownf * iota_col, axis=0, keepdims=True)
        last_e = jnp.max(jnp.where(nb > 0, iota_col, 0.0))
        be = jnp.where(bv > 0, be_own, last_e)
        nused = jnp.sum(nb)
        ones = jnp.full((1, 128), 1, jnp.int32)
        meta_ref[0:1, :] = be.astype(jnp.int32)
        meta_ref[1:2, :] = bv.astype(jnp.int32)
        meta_ref[2:3, :] = ones * nused.astype(jnp.int32)
        meta_ref[3:4, :] = jnp.sum(
            jnp.where(lax.broadcasted_iota(jnp.int32, (E, 128), 0)
                      == lax.broadcasted_iota(jnp.int32, (E, 128), 1) % E,
                      ao, 0.0), axis=0, keepdims=True).astype(jnp.int32)


def _route(x, wg):
    return pl.pallas_call(
        _route_body,
        grid=(NCHUNK + 1,),
        in_specs=[
            pl.BlockSpec((GATE_CHUNK, D),
                         lambda c: (jnp.minimum(c, NCHUNK - 1), 0)),
            pl.BlockSpec((E, D), lambda c: (0, 0)),
        ],
        out_specs=[
            pl.BlockSpec((1, 1, GATE_CHUNK),
                         lambda c: (jnp.minimum(c, NCHUNK - 1), 0, 0)),
            pl.BlockSpec((1, 1, GATE_CHUNK),
                         lambda c: (jnp.minimum(c, NCHUNK - 1), 0, 0)),
            pl.BlockSpec((E, 128), lambda c: (0, 0)),
        ],
        out_shape=[
            jax.ShapeDtypeStruct((NCHUNK, 1, GATE_CHUNK), jnp.int32),
            jax.ShapeDtypeStruct((NCHUNK, 1, GATE_CHUNK), jnp.int32),
            jax.ShapeDtypeStruct((E, 128), jnp.int32),
        ],
        scratch_shapes=[
            pltpu.VMEM((E, 128), jnp.float32),
            pltpu.VMEM((E, 128), jnp.float32),
        ],
        compiler_params=pltpu.CompilerParams(
            dimension_semantics=("arbitrary",)),
    )(x, wg)


_SC_CHUNK = 16
_NW = 32
_TPW = N_TOK // _NW
_NCK = _TPW // _SC_CHUNK
_NBUF = 4
_LANES = 16


def _sc_dispatch_body(x_hbm, sel_hbm, rank_hbm, meta_hbm, xs_hbm,
                      aoff_v, sel_v, rank_v, idx_v, rows_v, gsem, ssem):
    wid = lax.axis_index("s") * 2 + lax.axis_index("c")
    base = wid * _TPW
    c0 = base // GATE_CHUNK
    off = base % GATE_CHUNK
    pltpu.sync_copy(meta_hbm.at[3, pl.ds(0, _LANES)], aoff_v)
    pltpu.sync_copy(sel_hbm.at[c0, 0, pl.ds(off, _TPW)], sel_v)
    pltpu.sync_copy(rank_hbm.at[c0, 0, pl.ds(off, _TPW)], rank_v)
    for j in range(_TPW // _LANES):
        sl = sel_v[pl.ds(j * _LANES, _LANES)]
        rk = rank_v[pl.ds(j * _LANES, _LANES)]
        row, col = divmod(j * _LANES, _SC_CHUNK)
        idx_v[row, pl.ds(col, _LANES)] = plsc.load_gather(aoff_v, [sl]) + rk
    gathers = [None] * _NCK
    scatters = [None] * _NCK

    def gather(k):
        return pltpu.async_copy(
            x_hbm.at[pl.ds(base + k * _SC_CHUNK, _SC_CHUNK)],
            rows_v.at[k % _NBUF], gsem[k % _NBUF])

    def scatter(k):
        return pltpu.async_copy(
            rows_v.at[k % _NBUF],
            xs_hbm.at[idx_v.at[k]],
            ssem[k % _NBUF])

    for k in range(_NBUF - 1):
        gathers[k] = gather(k)
    for k in range(_NCK):
        j = k + _NBUF - 1
        if j < _NCK:
            if k >= 1:
                scatters[k - 1].wait()
            gathers[j] = gather(j)
        gathers[k].wait()
        scatters[k] = scatter(k)
    for k in range(_NCK - _NBUF, _NCK):
        scatters[k].wait()


def _dispatch_rows(x, sel3, rank3, meta):
    mesh = plsc.VectorSubcoreMesh(core_axis_name="c", subcore_axis_name="s",
                                  num_cores=2, num_subcores=16)
    return pl.kernel(
        _sc_dispatch_body,
        out_type=jax.ShapeDtypeStruct((NP_PAD, D), jnp.float32),
        mesh=mesh,
        scratch_types=[
            pltpu.VMEM((_LANES,), jnp.int32),
            pltpu.VMEM((_TPW,), jnp.int32),
            pltpu.VMEM((_TPW,), jnp.int32),
            pltpu.VMEM((_NCK, _SC_CHUNK), jnp.int32),
            pltpu.VMEM((_NBUF, _SC_CHUNK, D), jnp.float32),
            [pltpu.SemaphoreType.DMA] * _NBUF,
            [pltpu.SemaphoreType.DMA] * _NBUF,
        ],
        compiler_params=pltpu.CompilerParams(needs_layout_passes=False),
    )(x, sel3, rank3, meta)


def _moe_body(meta_ref, xs_ref, w1_ref, b1_ref, w2_ref,
              um_ref, num, den, mref, tot):
    i = pl.program_id(0)
    v = meta_ref[1, i]

    @pl.when(i == 0)
    def _():
        num[...] = jnp.zeros_like(num)
        den[...] = jnp.zeros_like(den)
        tot[...] = jnp.zeros_like(tot)
        mref[...] = jnp.full_like(mref, NEG)

    @pl.when(v > 0)
    def _():
        e = meta_ref[0, i]
        vm = lax.broadcasted_iota(jnp.int32, (BLK, 1), 0) < v
        xb = jnp.where(vm, xs_ref[...], 0.0)
        h = jnp.tanh(
            lax.dot_general(xb, w1_ref[0], (((1,), (1,)), ((), ())),
                            preferred_element_type=jnp.float32)
            + b1_ref[0])
        s = lax.dot_general(h, w2_ref[0], (((1,), (1,)), ((), ())),
                            preferred_element_type=jnp.float32)
        s = jnp.where(vm, s, NEG)
        bm = jnp.max(s)
        ohe = lax.broadcasted_iota(jnp.int32, (E, 1), 0) == e
        m_old = mref[:, 0:1]
        m_old_e = jnp.max(jnp.where(ohe, m_old, NEG))
        m_new_e = jnp.maximum(m_old_e, bm)
        m_new = jnp.where(ohe, m_new_e, m_old)
        scale = jnp.exp(m_old - m_new)
        w = jnp.where(vm, jnp.exp(s - m_new_e), 0.0)
        den[:, 0:1] = den[:, 0:1] * scale + jnp.where(ohe, jnp.sum(w), 0.0)
        vmf = vm.astype(jnp.float32)
        wv = jnp.concatenate([w, vmf], axis=1)
        red = lax.dot_general(wv, xb, (((0,), (0,)), ((), ())),
                              preferred_element_type=jnp.float32)
        num[...] = num[...] * scale + jnp.where(ohe, red[0:1, :], 0.0)
        mref[:, 0:1] = m_new
        tot[0:1, :] += red[1:2, :]

    @pl.when(i == NB_MAX - 1)
    def _():
        d_ = den[:, 0:1]
        dsafe = jnp.where(d_ > 0, d_, 1.0)
        um_ref[...] = jnp.where(d_ > 0, num[...] / dsafe,
                                tot[0:1, :] / float(N_TOK))


def _moe_grouped(xs, w1, b1, w2, meta):
    grid_spec = pltpu.PrefetchScalarGridSpec(
        num_scalar_prefetch=1,
        grid=(NB_MAX,),
        in_specs=[
            pl.BlockSpec((BLK, D),
                         lambda i, m: (jnp.minimum(i, m[2, 0] - 1), 0)),
            pl.BlockSpec((1, H, D), lambda i, m: (m[0, i], 0, 0)),
            pl.BlockSpec((1, 1, H), lambda i, m: (m[0, i], 0, 0)),
            pl.BlockSpec((1, 1, H), lambda i, m: (m[0, i], 0, 0)),
        ],
        out_specs=pl.BlockSpec((E, D), lambda i, m: (0, 0)),
        scratch_shapes=[
            pltpu.VMEM((E, D), jnp.float32),
            pltpu.VMEM((E, 128), jnp.float32),
            pltpu.VMEM((E, 128), jnp.float32),
            pltpu.VMEM((E, D), jnp.float32),
        ],
    )
    return pl.pallas_call(
        _moe_body,
        grid_spec=grid_spec,
        out_shape=jax.ShapeDtypeStruct((E, D), jnp.float32),
        compiler_params=pltpu.CompilerParams(
            dimension_semantics=("arbitrary",)),
    )(meta, xs, w1, b1.reshape(E, 1, H), w2.reshape(E, 1, H))


def kernel(inputs, Wg, W1, b1, W2, b2):
    del b2
    sel3, rank3, meta = _route(inputs, Wg)
    xs = _dispatch_rows(inputs, sel3, rank3, meta)
    um = _moe_grouped(xs, W1, b1, W2.reshape(E, H), meta)
    return um.reshape(1, E * D)


# speedup vs baseline: 1.0210x; 1.0210x over previous
import functools

import jax
import jax.numpy as jnp
from jax import lax
from jax.experimental import pallas as pl
from jax.experimental.pallas import tpu as pltpu
from jax.experimental.pallas import tpu_sc as plsc

N_TOK = 8192
D = 1024
H = 2048
E = 8
BLK = 512
NB_MAX = N_TOK // BLK + E
NP_PAD = NB_MAX * BLK
GATE_CHUNK = 1024
NCHUNK = N_TOK // GATE_CHUNK
NEG = -1e30


def _route_body(x_ref, wg_ref, sel_ref, rank_ref, meta_ref, cnt, carry):
    c = pl.program_id(0)

    @pl.when(c == 0)
    def _():
        cnt[...] = jnp.zeros_like(cnt)
        carry[...] = jnp.zeros_like(carry)

    @pl.when(c < NCHUNK)
    def _():
        logits = lax.dot_general(x_ref[...], wg_ref[...],
                                 (((1,), (1,)), ((), ())),
                                 preferred_element_type=jnp.float32)
        mx = jnp.max(logits, axis=1, keepdims=True)
        lane = lax.broadcasted_iota(jnp.int32, (GATE_CHUNK, E), 1)
        bidx = jnp.min(jnp.where(logits == mx, lane, E),
                       axis=1, keepdims=True)
        sel_ref[...] = bidx.reshape(1, 1, GATE_CHUNK)
        sel2 = bidx.reshape(1, GATE_CHUNK)
        iota_e = lax.broadcasted_iota(jnp.int32, (E, GATE_CHUNK), 0)
        oht = (iota_e == sel2).astype(jnp.float32)
        scan = jnp.pad(oht[:, :-1], ((0, 0), (1, 0)))
        sh = 1
        while sh < GATE_CHUNK:
            scan = scan + jnp.pad(scan[:, :-sh], ((0, 0), (sh, 0)))
            sh *= 2
        rank = carry[:, 0:1] + scan
        rankf = jnp.sum(oht * rank, axis=0, keepdims=True)
        rank_ref[...] = rankf.astype(jnp.int32).reshape(1, 1, GATE_CHUNK)
        carry[:, 0:1] += jnp.sum(oht, axis=1, keepdims=True)
        cnt[:, 0:1] += jnp.sum(oht, axis=1, keepdims=True)

    @pl.when(c == NCHUNK)
    def _():
        counts = cnt[:, 0:1]
        nb = jnp.ceil(counts / BLK)
        tri = (lax.broadcasted_iota(jnp.int32, (E, E), 1)
               < lax.broadcasted_iota(jnp.int32, (E, E), 0)).astype(jnp.float32)
        ao = jnp.sum(tri * jnp.transpose(nb), axis=1, keepdims=True) * BLK
        blk_start = ao / BLK
        bidx = lax.broadcasted_iota(jnp.int32, (1, 128), 1).astype(jnp.float32)
        inb = bidx >= blk_start
        own = inb & (bidx < blk_start + nb)
        ownf = own.astype(jnp.float32)
        ve = jnp.clip(counts - (bidx - blk_start) * BLK, 0.0, float(BLK))
        bv = jnp.sum(ownf * ve, axis=0, keepdims=True)
        iota_col = lax.broadcasted_iota(jnp.int32, (E, 1), 0).astype(jnp.float32)
        be_own = jnp.sum(ownf * iota_col, axis=0, keepdims=True)
        last_e = jnp.max(jnp.where(nb > 0, iota_col, 0.0))
        be = jnp.where(bv > 0, be_own, last_e)
        nused = jnp.sum(nb)
        ones = jnp.full((1, 128), 1, jnp.int32)
        meta_ref[0:1, :] = be.astype(jnp.int32)
        meta_ref[1:2, :] = bv.astype(jnp.int32)
        meta_ref[2:3, :] = ones * nused.astype(jnp.int32)
        meta_ref[3:4, :] = jnp.sum(
            jnp.where(lax.broadcasted_iota(jnp.int32, (E, 128), 0)
                      == lax.broadcasted_iota(jnp.int32, (E, 128), 1) % E,
                      ao, 0.0), axis=0, keepdims=True).astype(jnp.int32)


def _route(x, wg):
    return pl.pallas_call(
        _route_body,
        grid=(NCHUNK + 1,),
        in_specs=[
            pl.BlockSpec((GATE_CHUNK, D),
                         lambda c: (jnp.minimum(c, NCHUNK - 1), 0)),
            pl.BlockSpec((E, D), lambda c: (0, 0)),
        ],
        out_specs=[
            pl.BlockSpec((1, 1, GATE_CHUNK),
                         lambda c: (jnp.minimum(c, NCHUNK - 1), 0, 0)),
            pl.BlockSpec((1, 1, GATE_CHUNK),
                         lambda c: (jnp.minimum(c, NCHUNK - 1), 0, 0)),
            pl.BlockSpec((E, 128), lambda c: (0, 0)),
        ],
        out_shape=[
            jax.ShapeDtypeStruct((NCHUNK, 1, GATE_CHUNK), jnp.int32),
            jax.ShapeDtypeStruct((NCHUNK, 1, GATE_CHUNK), jnp.int32),
            jax.ShapeDtypeStruct((E, 128), jnp.int32),
        ],
        scratch_shapes=[
            pltpu.VMEM((E, 128), jnp.float32),
            pltpu.VMEM((E, 128), jnp.float32),
        ],
        compiler_params=pltpu.CompilerParams(
            dimension_semantics=("arbitrary",)),
    )(x, wg)


_SC_CHUNK = 16
_NW = 32
_TPW = N_TOK // _NW
_NCK = _TPW // _SC_CHUNK
_NBUF = 4
_LANES = 16


def _sc_dispatch_body(x_hbm, sel_hbm, rank_hbm, meta_hbm, xs_hbm,
                      aoff_v, sel_v, rank_v, idx_v, rows_v, gsem, ssem):
    wid = lax.axis_index("s") * 2 + lax.axis_index("c")
    base = wid * _TPW
    c0 = base // GATE_CHUNK
    off = base % GATE_CHUNK
    pltpu.sync_copy(meta_hbm.at[3, pl.ds(0, _LANES)], aoff_v)
    pltpu.sync_copy(sel_hbm.at[c0, 0, pl.ds(off, _TPW)], sel_v)
    pltpu.sync_copy(rank_hbm.at[c0, 0, pl.ds(off, _TPW)], rank_v)
    for j in range(_TPW // _LANES):
        sl = sel_v[pl.ds(j * _LANES, _LANES)]
        rk = rank_v[pl.ds(j * _LANES, _LANES)]
        row, col = divmod(j * _LANES, _SC_CHUNK)
        idx_v[row, pl.ds(col, _LANES)] = plsc.load_gather(aoff_v, [sl]) + rk
    gathers = [None] * _NCK
    scatters = [None] * _NCK

    def gather(k):
        return pltpu.async_copy(
            x_hbm.at[pl.ds(base + k * _SC_CHUNK, _SC_CHUNK)],
            rows_v.at[k % _NBUF], gsem[k % _NBUF])

    def scatter(k):
        return pltpu.async_copy(
            rows_v.at[k % _NBUF],
            xs_hbm.at[idx_v.at[k]],
            ssem[k % _NBUF])

    for k in range(_NBUF - 1):
        gathers[k] = gather(k)
    for k in range(_NCK):
        j = k + _NBUF - 1
        if j < _NCK:
            if k >= 1:
                scatters[k - 1].wait()
            gathers[j] = gather(j)
        gathers[k].wait()
        scatters[k] = scatter(k)
    for k in range(_NCK - _NBUF, _NCK):
        scatters[k].wait()


def _dispatch_rows(x, sel3, rank3, meta):
    mesh = plsc.VectorSubcoreMesh(core_axis_name="c", subcore_axis_name="s",
                                  num_cores=2, num_subcores=16)
    return pl.kernel(
        _sc_dispatch_body,
        out_type=jax.ShapeDtypeStruct((NP_PAD, D), jnp.float32),
        mesh=mesh,
        scratch_types=[
            pltpu.VMEM((_LANES,), jnp.int32),
            pltpu.VMEM((_TPW,), jnp.int32),
            pltpu.VMEM((_TPW,), jnp.int32),
            pltpu.VMEM((_NCK, _SC_CHUNK), jnp.int32),
            pltpu.VMEM((_NBUF, _SC_CHUNK, D), jnp.float32),
            [pltpu.SemaphoreType.DMA] * _NBUF,
            [pltpu.SemaphoreType.DMA] * _NBUF,
        ],
        compiler_params=pltpu.CompilerParams(needs_layout_passes=False),
    )(x, sel3, rank3, meta)


def _moe_body(meta_ref, xs_ref, w1_ref, b1_ref, w2_ref,
              um_ref, num, den, mref, tot):
    i = pl.program_id(0)
    v = meta_ref[1, i]

    @pl.when(i == 0)
    def _():
        num[...] = jnp.zeros_like(num)
        den[...] = jnp.zeros_like(den)
        tot[...] = jnp.zeros_like(tot)
        mref[...] = jnp.full_like(mref, NEG)

    @pl.when(v > 0)
    def _():
        e = meta_ref[0, i]
        vm = lax.broadcasted_iota(jnp.int32, (BLK, 1), 0) < v
        xb = jnp.where(vm, xs_ref[...], 0.0)
        h = jnp.tanh(
            lax.dot_general(xb, w1_ref[0], (((1,), (1,)), ((), ())),
                            preferred_element_type=jnp.float32)
            + b1_ref[0])
        s = lax.dot_general(h, w2_ref[0], (((1,), (1,)), ((), ())),
                            preferred_element_type=jnp.float32)
        s = jnp.where(vm, s, NEG)
        bm = jnp.max(s)
        ohe = lax.broadcasted_iota(jnp.int32, (E, 1), 0) == e
        m_old = mref[:, 0:1]
        m_old_e = jnp.max(jnp.where(ohe, m_old, NEG))
        m_new_e = jnp.maximum(m_old_e, bm)
        m_new = jnp.where(ohe, m_new_e, m_old)
        scale = jnp.exp(m_old - m_new)
        w = jnp.where(vm, jnp.exp(s - m_new_e), 0.0)
        den[:, 0:1] = den[:, 0:1] * scale + jnp.where(ohe, jnp.sum(w), 0.0)
        vmf = vm.astype(jnp.float32)
        wv = jnp.concatenate([w, vmf], axis=1)
        red = lax.dot_general(wv, xb, (((0,), (0,)), ((), ())),
                              preferred_element_type=jnp.float32)
        num[...] = num[...] * scale + jnp.where(ohe, red[0:1, :], 0.0)
        mref[:, 0:1] = m_new
        tot[0:1, :] += red[1:2, :]

    @pl.when(i == NB_MAX - 1)
    def _():
        d_ = den[:, 0:1]
        dsafe = jnp.where(d_ > 0, d_, 1.0)
        um_ref[...] = jnp.where(d_ > 0, num[...] / dsafe,
                                tot[0:1, :] / float(N_TOK))


def _moe_grouped(xs, w1, b1, w2, meta):
    grid_spec = pltpu.PrefetchScalarGridSpec(
        num_scalar_prefetch=1,
        grid=(NB_MAX,),
        in_specs=[
            pl.BlockSpec((BLK, D),
                         lambda i, m: (jnp.minimum(i, m[2, 0] - 1), 0)),
            pl.BlockSpec((1, H, D), lambda i, m: (m[0, i], 0, 0)),
            pl.BlockSpec((1, 1, H), lambda i, m: (m[0, i], 0, 0)),
            pl.BlockSpec((1, 1, H), lambda i, m: (m[0, i], 0, 0)),
        ],
        out_specs=pl.BlockSpec((E, D), lambda i, m: (0, 0)),
        scratch_shapes=[
            pltpu.VMEM((E, D), jnp.float32),
            pltpu.VMEM((E, 128), jnp.float32),
            pltpu.VMEM((E, 128), jnp.float32),
            pltpu.VMEM((E, D), jnp.float32),
        ],
    )
    return pl.pallas_call(
        _moe_body,
        grid_spec=grid_spec,
        out_shape=jax.ShapeDtypeStruct((E, D), jnp.float32),
        compiler_params=pltpu.CompilerParams(
            dimension_semantics=("arbitrary",)),
    )(meta, xs, w1, b1.reshape(E, 1, H), w2.reshape(E, 1, H))


def kernel(inputs, Wg, W1, b1, W2, b2):
    del b2
    sel3, rank3, meta = _route(inputs, Wg)
    xs = _dispatch_rows(inputs, sel3, rank3, meta)
    um = _moe_grouped(xs, W1, b1, W2.reshape(E, H), meta)
    return um.reshape(1, E * D)
